# baseline (speedup 1.0000x reference)
import functools

import jax
import jax.numpy as jnp
from jax.experimental import pallas as pl
from jax.experimental.pallas import tpu as pltpu

N = 1024
D = 64
TOPK = 10
HALF = 5


def _loss_kernel(s_ref, t_ref, idxr_ref, idxc_ref, out_ref):
    f32 = jnp.float32
    s = s_ref[...]
    t = t_ref[...]

    s = s / jnp.maximum(jnp.sqrt(jnp.sum(s * s, axis=1, keepdims=True)), 1e-12)
    t = t / jnp.maximum(jnp.sqrt(jnp.sum(t * t, axis=1, keepdims=True)), 1e-12)

    s_sq = jnp.sum(s * s, axis=1, keepdims=True)
    gs = jax.lax.dot_general(s, s, (((1,), (1,)), ((), ())),
                             preferred_element_type=f32)
    d2_s = s_sq + jnp.transpose(s_sq) - 2.0 * gs
    S = jnp.sqrt(jnp.maximum(d2_s, 0.0))
    mu_inv = 1.0 / jnp.mean(S, axis=1, keepdims=True)

    t_sq = jnp.sum(t * t, axis=1, keepdims=True)
    gt = jax.lax.dot_general(t, t, (((1,), (1,)), ((), ())),
                             preferred_element_type=f32)
    d2_t = t_sq + jnp.transpose(t_sq) - 2.0 * gt
    W_P = jnp.exp(-jnp.maximum(d2_t, 0.0))

    same = jnp.broadcast_to(idxr_ref[...], (N, N)) == jnp.broadcast_to(
        idxc_ref[...], (N, N))
    A = jnp.where(same, 1.0, W_P)

    col = jax.lax.broadcasted_iota(jnp.int32, (N, N), 1)
    for k in range(TOPK):
        sel = jnp.argmax(A, axis=1, keepdims=True)
        A = jnp.where(col == sel, -(k + 1.0), A)

    w_nn = (A < 0.0)
    h_sum = jnp.logical_and(A > -5.5, w_nn).astype(f32)

    bf16 = jnp.bfloat16
    w_nn_b = w_nn.astype(bf16)
    V = w_nn_b * jnp.transpose(w_nn_b)
    inner = jax.lax.dot_general(V, V, (((1,), (1,)), ((), ())),
                                preferred_element_type=f32)
    U = V * inner.astype(bf16)
    nnz = jnp.sum(V.astype(f32), axis=0, keepdims=True)
    G = (h_sum * (0.05 / jnp.maximum(nnz, 1.0))).astype(bf16)
    M = jax.lax.dot_general(G, U, (((1,), (1,)), ((), ())),
                            preferred_element_type=f32)
    W = 0.5 * W_P + M + jnp.transpose(M)

    row = jax.lax.broadcasted_iota(jnp.int32, (N, N), 0)
    offdiag = (row != col).astype(f32)
    Sn = S * mu_inv
    push_base = jnp.maximum(1.0 - Sn, 0.0)
    term = (Sn * Sn * W + push_base * push_base * (1.0 - W)) * offdiag
    loss = jnp.sum(term) / (N * (N - 1))
    out_ref[...] = jnp.broadcast_to(loss, (1, 1))


@functools.partial(jax.jit, static_argnames=())
def _run(s_emb, t_emb, idx):
    idxf = idx.astype(jnp.float32)
    out = pl.pallas_call(
        _loss_kernel,
        out_shape=jax.ShapeDtypeStruct((1, 1), jnp.float32),
        compiler_params=pltpu.CompilerParams(
            vmem_limit_bytes=128 * 1024 * 1024),
    )(s_emb, t_emb, idxf.reshape(N, 1), idxf.reshape(1, N))
    return out[0, 0]


def kernel(s_emb, t_emb, idx):
    return _run(s_emb, t_emb, idx)


# speedup vs baseline: 1.0661x; 1.0661x over previous
import functools

import jax
import jax.numpy as jnp
from jax.experimental import pallas as pl
from jax.experimental.pallas import tpu as pltpu

N = 1024
D = 64
TOPK = 10
HALF = 5


def _loss_kernel(s_ref, t_ref, idxc_ref, out_ref):
    f32 = jnp.float32
    s = s_ref[...]
    t = t_ref[...]

    s = s / jnp.maximum(jnp.sqrt(jnp.sum(s * s, axis=1, keepdims=True)), 1e-12)
    t = t / jnp.maximum(jnp.sqrt(jnp.sum(t * t, axis=1, keepdims=True)), 1e-12)

    s_sq = jnp.sum(s * s, axis=1, keepdims=True)
    gs = jax.lax.dot_general(s, s, (((1,), (1,)), ((), ())),
                             preferred_element_type=f32)
    d2_s = s_sq + jnp.transpose(s_sq) - 2.0 * gs
    S = jnp.sqrt(jnp.maximum(d2_s, 0.0))
    mu_inv = 1.0 / jnp.mean(S, axis=1, keepdims=True)

    t_sq = jnp.sum(t * t, axis=1, keepdims=True)
    gt = jax.lax.dot_general(t, t, (((1,), (1,)), ((), ())),
                             preferred_element_type=f32)
    d2_t = t_sq + jnp.transpose(t_sq) - 2.0 * gt
    W_P = jnp.exp(-jnp.maximum(d2_t, 0.0))

    icf = idxc_ref[...].astype(f32)
    same = jnp.broadcast_to(jnp.transpose(icf), (N, N)) == jnp.broadcast_to(
        icf, (N, N))
    A = jnp.where(same, 1.0, W_P)

    col = jax.lax.broadcasted_iota(jnp.int32, (N, N), 1)
    for k in range(TOPK):
        sel = jnp.argmax(A, axis=1, keepdims=True)
        A = jnp.where(col == sel, -(k + 1.0), A)

    w_nn = (A < 0.0)
    h_sum = jnp.logical_and(A > -5.5, w_nn).astype(f32)

    bf16 = jnp.bfloat16
    w_nn_b = w_nn.astype(bf16)
    V = w_nn_b * jnp.transpose(w_nn_b)
    inner = jax.lax.dot_general(V, V, (((1,), (1,)), ((), ())),
                                preferred_element_type=f32)
    U = V * inner.astype(bf16)
    nnz = jnp.sum(V.astype(f32), axis=0, keepdims=True)
    G = (h_sum * (0.05 / jnp.maximum(nnz, 1.0))).astype(bf16)
    M = jax.lax.dot_general(G, U, (((1,), (1,)), ((), ())),
                            preferred_element_type=f32)
    W = 0.5 * W_P + M + jnp.transpose(M)

    row = jax.lax.broadcasted_iota(jnp.int32, (N, N), 0)
    offdiag = (row != col).astype(f32)
    Sn = S * mu_inv
    push_base = jnp.maximum(1.0 - Sn, 0.0)
    term = (Sn * Sn * W + push_base * push_base * (1.0 - W)) * offdiag
    loss = jnp.sum(term) / (N * (N - 1))
    out_ref[...] = jnp.broadcast_to(loss, (1, 1))


@functools.partial(jax.jit, static_argnames=())
def _run(s_emb, t_emb, idx):
    out = pl.pallas_call(
        _loss_kernel,
        out_shape=jax.ShapeDtypeStruct((1, 1), jnp.float32),
        compiler_params=pltpu.CompilerParams(
            vmem_limit_bytes=128 * 1024 * 1024),
    )(s_emb, t_emb, idx.reshape(1, N))
    return jnp.reshape(out, ())


def kernel(s_emb, t_emb, idx):
    return _run(s_emb, t_emb, idx)
